# baseline (speedup 1.0000x reference)
import jax
import jax.numpy as jnp
from jax import lax
from jax.experimental import pallas as pl
from jax.experimental.pallas import tpu as pltpu
from jax.experimental.pallas import tpu_sc as plsc

_EPS = 1e-5
_CHUNK = 128
_NW = 32


def _relu(x):
    return jnp.maximum(x, 0.0)


def _dot(a, b):
    return jnp.dot(a, b, preferred_element_type=jnp.float32)


def _ln(h, g, b):
    mu = jnp.mean(h, axis=-1, keepdims=True)
    var = jnp.mean((h - mu) ** 2, axis=-1, keepdims=True)
    return (h - mu) * lax.rsqrt(var + _EPS) * g + b


def _enc_body(x, W1, b1, W2, b2, g, be, o):
    h = _relu(_dot(x[...], W1[...]) + b1[...])
    h = _relu(_dot(h, W2[...]) + b2[...])
    o[...] = _ln(h, g[...], be[...])


def _pre_body(x, Ws, Wr, b1, a, b):
    xv = x[...]
    a[...] = _dot(xv, Ws[...]) + b1[...]
    b[...] = _dot(xv, Wr[...])


def _edge_body(c, e, We, W2, b2, g, be, eo):
    ev = e[...]
    h = _relu(c[...] + _dot(ev, We[...]))
    h = _relu(_dot(h, W2[...]) + b2[...])
    eo[...] = ev + _ln(h, g[...], be[...])


def _node_body(x, q0, q1, q2, q3, p0, p1, p2, p3,
               Wn, Wa, b1, W2, b2, g, be, o):
    xv = x[...]
    aggr = ((q0[...] + q1[...]) + (q2[...] + q3[...])
            - (p0[...] + p1[...]) - (p2[...] + p3[...]))
    h = _relu(_dot(xv, Wn[...]) + _dot(aggr, Wa[...]) + b1[...])
    h = _relu(_dot(h, W2[...]) + b2[...])
    o[...] = xv + _ln(h, g[...], be[...])


def _dec_body(x, W1, b1, W2, b2, dt, o):
    h = _dot(x[...], W1[...]) + b1[...]
    h = h * jax.nn.sigmoid(h)
    o[...] = (_dot(h, W2[...]) + b2[...]) * dt[...]


def _pick_tr(rows):
    for t in (2000, 1000, 500, 200, 100, 8):
        if rows % t == 0:
            return t
    return rows


def _rowwise(body, tiled_in, const_in, out_widths, out_dtype=jnp.float32):
    rows = tiled_in[0].shape[0]
    tr = _pick_tr(rows)
    grid = rows // tr
    in_specs = (
        [pl.BlockSpec((tr, a.shape[1]), lambda i: (i, 0)) for a in tiled_in]
        + [pl.BlockSpec(a.shape, lambda i, _nd=a.ndim: (0,) * _nd) for a in const_in]
    )
    out_shape = [jax.ShapeDtypeStruct((rows, w), out_dtype) for w in out_widths]
    out_specs = [pl.BlockSpec((tr, w), lambda i: (i, 0)) for w in out_widths]
    outs = pl.pallas_call(
        body,
        grid=(grid,),
        in_specs=in_specs,
        out_specs=out_specs,
        out_shape=out_shape,
    )(*tiled_in, *const_in)
    return outs


_SS = 256
_NBUF = 3


def _make_gather(n, e, l):
    nch = e // _SS
    groups = (((nch + _NW - 1) // _NW) + _NBUF - 1) // _NBUF
    mesh = plsc.VectorSubcoreMesh(core_axis_name="c", subcore_axis_name="s", num_cores=2, num_subcores=16)

    def body(a_hbm, b_hbm, snd_hbm, rcv_hbm, out_hbm,
             idx_s, idx_r, rows, sem_idx, sem_g, sem_out):
        cid = lax.axis_index("c")
        sid = lax.axis_index("s")
        wid = sid * 2 + cid

        def issue_idx(k, b):
            c = wid + _NW * k

            @pl.when(c < nch)
            def _():
                base = c * _SS
                pltpu.async_copy(snd_hbm.at[pl.ds(base, _SS)], idx_s[b], sem_idx[b])
                pltpu.async_copy(rcv_hbm.at[pl.ds(base, _SS)], idx_r[b], sem_idx[b])

        for b in range(_NBUF):
            issue_idx(b, b)

        def step(k, b):
            c = wid + _NW * k

            @pl.when(c < nch)
            def _():
                base = c * _SS
                pltpu.make_async_copy(snd_hbm.at[pl.ds(base, _SS)], idx_s[b], sem_idx[b]).wait()
                pltpu.make_async_copy(rcv_hbm.at[pl.ds(base, _SS)], idx_r[b], sem_idx[b]).wait()

                @pl.when(k >= _NBUF)
                def _():
                    pltpu.make_async_copy(rows[b], out_hbm.at[pl.ds(base, _SS), :], sem_out[b]).wait()

                d0 = pltpu.async_copy(a_hbm.at[idx_s[b].at[pl.ds(0, _CHUNK)]],
                                      rows[b].at[pl.ds(0, _CHUNK), :], sem_g[b])
                d1 = pltpu.async_copy(a_hbm.at[idx_s[b].at[pl.ds(_CHUNK, _CHUNK)]],
                                      rows[b].at[pl.ds(_CHUNK, _CHUNK), :], sem_g[b])
                d0.wait()
                d1.wait()
                d2 = pltpu.async_copy(b_hbm.at[idx_r[b].at[pl.ds(0, _CHUNK)]],
                                      rows[b].at[pl.ds(0, _CHUNK), :], sem_g[b], add=True)
                d3 = pltpu.async_copy(b_hbm.at[idx_r[b].at[pl.ds(_CHUNK, _CHUNK)]],
                                      rows[b].at[pl.ds(_CHUNK, _CHUNK), :], sem_g[b], add=True)
                d2.wait()
                d3.wait()

                issue_idx(k + _NBUF, b)
                pltpu.async_copy(rows[b], out_hbm.at[pl.ds(base, _SS), :], sem_out[b])

        def group(g, carry):
            for b in range(_NBUF):
                step(g * _NBUF + b, b)
            return carry

        lax.fori_loop(0, groups, group, 0)
        for b in range(_NBUF):
            pltpu.make_async_copy(rows[b], out_hbm.at[pl.ds(0, _SS), :], sem_out[b]).wait()

    return pl.kernel(
        body,
        out_type=jax.ShapeDtypeStruct((e, l), jnp.float32),
        mesh=mesh,
        scratch_types=[
            [pltpu.VMEM((_SS,), jnp.int32) for _ in range(_NBUF)],
            [pltpu.VMEM((_SS,), jnp.int32) for _ in range(_NBUF)],
            [pltpu.VMEM((_SS, l), jnp.float32) for _ in range(_NBUF)],
            [pltpu.SemaphoreType.DMA for _ in range(_NBUF)],
            [pltpu.SemaphoreType.DMA for _ in range(_NBUF)],
            [pltpu.SemaphoreType.DMA for _ in range(_NBUF)],
        ],
    )


def _make_scatter(n, e, l):
    rpt = ((n // 16) // 8) * 8
    last = n - 15 * rpt
    mesh = plsc.VectorSubcoreMesh(core_axis_name="c", subcore_axis_name="s", num_cores=2, num_subcores=16)

    def _striped_copy(sid, src, dst):
        @pl.when(sid < 15)
        def _():
            pltpu.sync_copy(src.at[pl.ds(sid * rpt, rpt), :],
                            dst.at[pl.ds(sid * rpt, rpt), :])

        @pl.when(sid == 15)
        def _():
            pltpu.sync_copy(src.at[pl.ds(15 * rpt, last), :],
                            dst.at[pl.ds(15 * rpt, last), :])

    nch = e // _CHUNK
    nbuf = 3
    groups = (((nch + _NW - 1) // _NW) + nbuf - 1) // nbuf

    def body(ne_hbm, rcv_hbm, zero_hbm, out_hbm, idx_r, rows, sem_in, sem_sc, acc):
        cid = lax.axis_index("c")
        sid = lax.axis_index("s")
        wid = sid * 2 + cid

        def issue_in(k, b):
            c = wid + _NW * k

            @pl.when(c < nch)
            def _():
                base = c * _CHUNK
                pltpu.async_copy(rcv_hbm.at[pl.ds(base, _CHUNK)], idx_r[b], sem_in[b])
                pltpu.async_copy(ne_hbm.at[pl.ds(base, _CHUNK), :], rows[b], sem_in[b])

        for b in range(2):
            issue_in(b, b)

        _striped_copy(sid, zero_hbm, acc)
        plsc.subcore_barrier()

        def step(k, b):
            c = wid + _NW * k

            @pl.when(c < nch)
            def _():
                base = c * _CHUNK
                pltpu.make_async_copy(rcv_hbm.at[pl.ds(base, _CHUNK)], idx_r[b], sem_in[b]).wait()
                pltpu.make_async_copy(ne_hbm.at[pl.ds(base, _CHUNK), :], rows[b], sem_in[b]).wait()
                pltpu.async_copy(rows[b], acc.at[idx_r[b]], sem_sc[b], add=True)
                bp = (b + 2) % nbuf

                @pl.when(k >= 1)
                def _():
                    pltpu.make_async_copy(rows[bp], acc.at[idx_r[bp]], sem_sc[bp]).wait()

                issue_in(k + 2, bp)

        def group(g, carry):
            for b in range(nbuf):
                step(g * nbuf + b, b)
            return carry

        lax.fori_loop(0, groups, group, 0)
        m = (nch - wid + _NW - 1) // _NW
        for b in range(nbuf):
            @pl.when((m - 1) % nbuf == b)
            def _(b=b):
                pltpu.make_async_copy(rows[b], acc.at[idx_r[b]], sem_sc[b]).wait()

        plsc.subcore_barrier()
        _striped_copy(sid, acc, out_hbm.at[cid])

    return pl.kernel(
        body,
        out_type=jax.ShapeDtypeStruct((2, n, l), jnp.float32),
        mesh=mesh,
        scratch_types=[
            [pltpu.VMEM((_CHUNK,), jnp.int32) for _ in range(3)],
            [pltpu.VMEM((_CHUNK, l), jnp.float32) for _ in range(3)],
            [pltpu.SemaphoreType.DMA for _ in range(3)],
            [pltpu.SemaphoreType.DMA for _ in range(3)],
            pltpu.VMEM_SHARED((n, l), jnp.float32),
        ],
    )


def kernel(node_features, mesh_edge_features, senders, receivers, params):
    p = params
    n, l = node_features.shape
    e = senders.shape[0]
    s_blocks = p["blk_e_W1"].shape[0]

    def r2(v):
        return v.reshape(1, -1)

    node_lat, = _rowwise(
        _enc_body, [node_features],
        [p["enc_n_W1"], r2(p["enc_n_b1"]), p["enc_n_W2"], r2(p["enc_n_b2"]),
         r2(p["enc_n_g"]), r2(p["enc_n_be"])], [l])

    e0 = (e * 3 // 5 // 2000) * 2000
    if e0 <= 0 or e0 >= e:
        e0 = e // 2
    bounds = (0, e0, e)
    eh_sizes = (e0, e - e0)
    snd_h = tuple(senders[bounds[h]:bounds[h + 1]] for h in range(2))
    rcv_h = tuple(receivers[bounds[h]:bounds[h + 1]] for h in range(2))
    edge_lat_h = []
    for h in range(2):
        el, = _rowwise(
            _enc_body, [mesh_edge_features[bounds[h]:bounds[h + 1]]],
            [p["enc_e_W1"], r2(p["enc_e_b1"]), p["enc_e_W2"], r2(p["enc_e_b2"]),
             r2(p["enc_e_g"]), r2(p["enc_e_be"])], [l])
        edge_lat_h.append(el)

    gather_h = tuple(_make_gather(n, eh_sizes[h], l) for h in range(2))
    scatter_h = tuple(_make_scatter(n, eh_sizes[h], l) for h in range(2))
    zeros_nl = jnp.zeros((n, l), jnp.float32)

    prev_h = [scatter_h[h](edge_lat_h[h], rcv_h[h], zeros_nl) for h in range(2)]
    for s in range(s_blocks):
        w1 = p["blk_e_W1"][s]
        a_tab, b_tab = _rowwise(
            _pre_body, [node_lat],
            [w1[:l], w1[l:2 * l], r2(p["blk_e_b1"][s])], [l, l])
        cur_h = []
        for h in range(2):
            c = gather_h[h](a_tab, b_tab, snd_h[h], rcv_h[h])
            edge_lat_h[h], = _rowwise(
                _edge_body, [c, edge_lat_h[h]],
                [w1[2 * l:], p["blk_e_W2"][s], r2(p["blk_e_b2"][s]),
                 r2(p["blk_e_g"][s]), r2(p["blk_e_be"][s])], [l])
            cur_h.append(scatter_h[h](edge_lat_h[h], rcv_h[h], zeros_nl))
        wn1 = p["blk_n_W1"][s]
        node_lat, = _rowwise(
            _node_body, [node_lat, cur_h[0][0], cur_h[0][1],
                         cur_h[1][0], cur_h[1][1],
                         prev_h[0][0], prev_h[0][1],
                         prev_h[1][0], prev_h[1][1]],
            [wn1[:l], wn1[l:], r2(p["blk_n_b1"][s]), p["blk_n_W2"][s],
             r2(p["blk_n_b2"][s]), r2(p["blk_n_g"][s]), r2(p["blk_n_be"][s])],
            [l])
        prev_h = cur_h

    tw_out = p["dec_b2"].shape[0]
    tw = 5
    out_c = tw_out // tw
    dt = jnp.repeat(jnp.arange(1, tw + 1), out_c).astype(jnp.float32)
    dec, = _rowwise(
        _dec_body, [node_lat],
        [p["dec_W1"], r2(p["dec_b1"]), p["dec_W2"], r2(p["dec_b2"]), r2(dt)],
        [tw_out])
    return dec.reshape(n, tw, out_c).transpose(1, 0, 2)


# speedup vs baseline: 1.0001x; 1.0001x over previous
import jax
import jax.numpy as jnp
from jax import lax
from jax.experimental import pallas as pl
from jax.experimental.pallas import tpu as pltpu
from jax.experimental.pallas import tpu_sc as plsc

_EPS = 1e-5
_CHUNK = 128
_NW = 32


def _relu(x):
    return jnp.maximum(x, 0.0)


def _dot(a, b):
    return jnp.dot(a, b, preferred_element_type=jnp.float32)


def _ln(h, g, b):
    mu = jnp.mean(h, axis=-1, keepdims=True)
    var = jnp.mean((h - mu) ** 2, axis=-1, keepdims=True)
    return (h - mu) * lax.rsqrt(var + _EPS) * g + b


def _enc_body(x, W1, b1, W2, b2, g, be, o):
    h = _relu(_dot(x[...], W1[...]) + b1[...])
    h = _relu(_dot(h, W2[...]) + b2[...])
    o[...] = _ln(h, g[...], be[...])


def _pre_body(x, Ws, Wr, b1, a, b):
    xv = x[...]
    a[...] = _dot(xv, Ws[...]) + b1[...]
    b[...] = _dot(xv, Wr[...])


def _edge_body(c, e, We, W2, b2, g, be, eo):
    ev = e[...]
    h = _relu(c[...] + _dot(ev, We[...]))
    h = _relu(_dot(h, W2[...]) + b2[...])
    eo[...] = ev + _ln(h, g[...], be[...])


def _node_body(x, q0, q1, q2, q3, p0, p1, p2, p3,
               Wn, Wa, b1, W2, b2, g, be, o):
    xv = x[...]
    aggr = ((q0[...] + q1[...]) + (q2[...] + q3[...])
            - (p0[...] + p1[...]) - (p2[...] + p3[...]))
    h = _relu(_dot(xv, Wn[...]) + _dot(aggr, Wa[...]) + b1[...])
    h = _relu(_dot(h, W2[...]) + b2[...])
    o[...] = xv + _ln(h, g[...], be[...])


def _dec_body(x, W1, b1, W2, b2, dt, o):
    h = _dot(x[...], W1[...]) + b1[...]
    h = h * jax.nn.sigmoid(h)
    o[...] = (_dot(h, W2[...]) + b2[...]) * dt[...]


def _pick_tr(rows):
    for t in (2000, 1000, 500, 200, 100, 8):
        if rows % t == 0:
            return t
    return rows


def _rowwise(body, tiled_in, const_in, out_widths, out_dtype=jnp.float32):
    rows = tiled_in[0].shape[0]
    tr = _pick_tr(rows)
    grid = rows // tr
    in_specs = (
        [pl.BlockSpec((tr, a.shape[1]), lambda i: (i, 0)) for a in tiled_in]
        + [pl.BlockSpec(a.shape, lambda i, _nd=a.ndim: (0,) * _nd) for a in const_in]
    )
    out_shape = [jax.ShapeDtypeStruct((rows, w), out_dtype) for w in out_widths]
    out_specs = [pl.BlockSpec((tr, w), lambda i: (i, 0)) for w in out_widths]
    outs = pl.pallas_call(
        body,
        grid=(grid,),
        in_specs=in_specs,
        out_specs=out_specs,
        out_shape=out_shape,
    )(*tiled_in, *const_in)
    return outs


_SS = 256
_NBUF = 3


def _make_gather(n, e, l):
    nch = e // _SS
    groups = (((nch + _NW - 1) // _NW) + _NBUF - 1) // _NBUF
    mesh = plsc.VectorSubcoreMesh(core_axis_name="c", subcore_axis_name="s", num_cores=2, num_subcores=16)

    def body(a_hbm, b_hbm, snd_hbm, rcv_hbm, out_hbm,
             idx_s, idx_r, rows, sem_idx, sem_g, sem_out):
        cid = lax.axis_index("c")
        sid = lax.axis_index("s")
        wid = sid * 2 + cid

        def issue_idx(k, b):
            c = wid + _NW * k

            @pl.when(c < nch)
            def _():
                base = c * _SS
                pltpu.async_copy(snd_hbm.at[pl.ds(base, _SS)], idx_s[b], sem_idx[b])
                pltpu.async_copy(rcv_hbm.at[pl.ds(base, _SS)], idx_r[b], sem_idx[b])

        for b in range(_NBUF):
            issue_idx(b, b)

        def step(k, b):
            c = wid + _NW * k

            @pl.when(c < nch)
            def _():
                base = c * _SS
                pltpu.make_async_copy(snd_hbm.at[pl.ds(base, _SS)], idx_s[b], sem_idx[b]).wait()
                pltpu.make_async_copy(rcv_hbm.at[pl.ds(base, _SS)], idx_r[b], sem_idx[b]).wait()

                @pl.when(k >= _NBUF)
                def _():
                    pltpu.make_async_copy(rows[b], out_hbm.at[pl.ds(base, _SS), :], sem_out[b]).wait()

                d0 = pltpu.async_copy(a_hbm.at[idx_s[b].at[pl.ds(0, _CHUNK)]],
                                      rows[b].at[pl.ds(0, _CHUNK), :], sem_g[b])
                d1 = pltpu.async_copy(a_hbm.at[idx_s[b].at[pl.ds(_CHUNK, _CHUNK)]],
                                      rows[b].at[pl.ds(_CHUNK, _CHUNK), :], sem_g[b])
                d0.wait()
                d1.wait()
                d2 = pltpu.async_copy(b_hbm.at[idx_r[b].at[pl.ds(0, _CHUNK)]],
                                      rows[b].at[pl.ds(0, _CHUNK), :], sem_g[b], add=True)
                d3 = pltpu.async_copy(b_hbm.at[idx_r[b].at[pl.ds(_CHUNK, _CHUNK)]],
                                      rows[b].at[pl.ds(_CHUNK, _CHUNK), :], sem_g[b], add=True)
                d2.wait()
                d3.wait()

                issue_idx(k + _NBUF, b)
                pltpu.async_copy(rows[b], out_hbm.at[pl.ds(base, _SS), :], sem_out[b])

        def group(g, carry):
            for b in range(_NBUF):
                step(g * _NBUF + b, b)
            return carry

        lax.fori_loop(0, groups, group, 0)
        for b in range(_NBUF):
            pltpu.make_async_copy(rows[b], out_hbm.at[pl.ds(0, _SS), :], sem_out[b]).wait()

    return pl.kernel(
        body,
        out_type=jax.ShapeDtypeStruct((e, l), jnp.float32),
        mesh=mesh,
        scratch_types=[
            [pltpu.VMEM((_SS,), jnp.int32) for _ in range(_NBUF)],
            [pltpu.VMEM((_SS,), jnp.int32) for _ in range(_NBUF)],
            [pltpu.VMEM((_SS, l), jnp.float32) for _ in range(_NBUF)],
            [pltpu.SemaphoreType.DMA for _ in range(_NBUF)],
            [pltpu.SemaphoreType.DMA for _ in range(_NBUF)],
            [pltpu.SemaphoreType.DMA for _ in range(_NBUF)],
        ],
    )


def _make_scatter(n, e, l):
    rpt = ((n // 16) // 8) * 8
    last = n - 15 * rpt
    mesh = plsc.VectorSubcoreMesh(core_axis_name="c", subcore_axis_name="s", num_cores=2, num_subcores=16)

    def _striped_copy(sid, src, dst):
        @pl.when(sid < 15)
        def _():
            pltpu.sync_copy(src.at[pl.ds(sid * rpt, rpt), :],
                            dst.at[pl.ds(sid * rpt, rpt), :])

        @pl.when(sid == 15)
        def _():
            pltpu.sync_copy(src.at[pl.ds(15 * rpt, last), :],
                            dst.at[pl.ds(15 * rpt, last), :])

    nch = e // _CHUNK
    nbuf = 3
    groups = (((nch + _NW - 1) // _NW) + nbuf - 1) // nbuf

    def body(ne_hbm, rcv_hbm, zero_hbm, out_hbm, idx_r, rows, sem_in, sem_sc, acc):
        cid = lax.axis_index("c")
        sid = lax.axis_index("s")
        wid = sid * 2 + cid

        def issue_in(k, b):
            c = wid + _NW * k

            @pl.when(c < nch)
            def _():
                base = c * _CHUNK
                pltpu.async_copy(rcv_hbm.at[pl.ds(base, _CHUNK)], idx_r[b], sem_in[b])
                pltpu.async_copy(ne_hbm.at[pl.ds(base, _CHUNK), :], rows[b], sem_in[b])

        for b in range(2):
            issue_in(b, b)

        _striped_copy(sid, zero_hbm, acc)
        plsc.subcore_barrier()

        def step(k, b):
            c = wid + _NW * k

            @pl.when(c < nch)
            def _():
                base = c * _CHUNK
                pltpu.make_async_copy(rcv_hbm.at[pl.ds(base, _CHUNK)], idx_r[b], sem_in[b]).wait()
                pltpu.make_async_copy(ne_hbm.at[pl.ds(base, _CHUNK), :], rows[b], sem_in[b]).wait()
                pltpu.async_copy(rows[b], acc.at[idx_r[b]], sem_sc[b], add=True)
                bp = (b + 2) % nbuf

                @pl.when(k >= 1)
                def _():
                    pltpu.make_async_copy(rows[bp], acc.at[idx_r[bp]], sem_sc[bp]).wait()

                issue_in(k + 2, bp)

        def group(g, carry):
            for b in range(nbuf):
                step(g * nbuf + b, b)
            return carry

        lax.fori_loop(0, groups, group, 0)
        m = (nch - wid + _NW - 1) // _NW
        for b in range(nbuf):
            @pl.when((m - 1) % nbuf == b)
            def _(b=b):
                pltpu.make_async_copy(rows[b], acc.at[idx_r[b]], sem_sc[b]).wait()

        plsc.subcore_barrier()
        _striped_copy(sid, acc, out_hbm.at[cid])

    return pl.kernel(
        body,
        out_type=jax.ShapeDtypeStruct((2, n, l), jnp.float32),
        mesh=mesh,
        scratch_types=[
            [pltpu.VMEM((_CHUNK,), jnp.int32) for _ in range(3)],
            [pltpu.VMEM((_CHUNK, l), jnp.float32) for _ in range(3)],
            [pltpu.SemaphoreType.DMA for _ in range(3)],
            [pltpu.SemaphoreType.DMA for _ in range(3)],
            pltpu.VMEM_SHARED((n, l), jnp.float32),
        ],
    )


def kernel(node_features, mesh_edge_features, senders, receivers, params):
    p = params
    n, l = node_features.shape
    e = senders.shape[0]
    s_blocks = p["blk_e_W1"].shape[0]

    def r2(v):
        return v.reshape(1, -1)

    node_lat, = _rowwise(
        _enc_body, [node_features],
        [p["enc_n_W1"], r2(p["enc_n_b1"]), p["enc_n_W2"], r2(p["enc_n_b2"]),
         r2(p["enc_n_g"]), r2(p["enc_n_be"])], [l])

    e0 = (e * 2 // 5 // 2000) * 2000
    if e0 <= 0 or e0 >= e:
        e0 = e // 2
    bounds = (0, e0, e)
    eh_sizes = (e0, e - e0)
    snd_h = tuple(senders[bounds[h]:bounds[h + 1]] for h in range(2))
    rcv_h = tuple(receivers[bounds[h]:bounds[h + 1]] for h in range(2))
    edge_lat_h = []
    for h in range(2):
        el, = _rowwise(
            _enc_body, [mesh_edge_features[bounds[h]:bounds[h + 1]]],
            [p["enc_e_W1"], r2(p["enc_e_b1"]), p["enc_e_W2"], r2(p["enc_e_b2"]),
             r2(p["enc_e_g"]), r2(p["enc_e_be"])], [l])
        edge_lat_h.append(el)

    gather_h = tuple(_make_gather(n, eh_sizes[h], l) for h in range(2))
    scatter_h = tuple(_make_scatter(n, eh_sizes[h], l) for h in range(2))
    zeros_nl = jnp.zeros((n, l), jnp.float32)

    prev_h = [scatter_h[h](edge_lat_h[h], rcv_h[h], zeros_nl) for h in range(2)]
    for s in range(s_blocks):
        w1 = p["blk_e_W1"][s]
        a_tab, b_tab = _rowwise(
            _pre_body, [node_lat],
            [w1[:l], w1[l:2 * l], r2(p["blk_e_b1"][s])], [l, l])
        cur_h = []
        for h in range(2):
            c = gather_h[h](a_tab, b_tab, snd_h[h], rcv_h[h])
            edge_lat_h[h], = _rowwise(
                _edge_body, [c, edge_lat_h[h]],
                [w1[2 * l:], p["blk_e_W2"][s], r2(p["blk_e_b2"][s]),
                 r2(p["blk_e_g"][s]), r2(p["blk_e_be"][s])], [l])
            cur_h.append(scatter_h[h](edge_lat_h[h], rcv_h[h], zeros_nl))
        wn1 = p["blk_n_W1"][s]
        node_lat, = _rowwise(
            _node_body, [node_lat, cur_h[0][0], cur_h[0][1],
                         cur_h[1][0], cur_h[1][1],
                         prev_h[0][0], prev_h[0][1],
                         prev_h[1][0], prev_h[1][1]],
            [wn1[:l], wn1[l:], r2(p["blk_n_b1"][s]), p["blk_n_W2"][s],
             r2(p["blk_n_b2"][s]), r2(p["blk_n_g"][s]), r2(p["blk_n_be"][s])],
            [l])
        prev_h = cur_h

    tw_out = p["dec_b2"].shape[0]
    tw = 5
    out_c = tw_out // tw
    dt = jnp.repeat(jnp.arange(1, tw + 1), out_c).astype(jnp.float32)
    dec, = _rowwise(
        _dec_body, [node_lat],
        [p["dec_W1"], r2(p["dec_b1"]), p["dec_W2"], r2(p["dec_b2"]), r2(dt)],
        [tw_out])
    return dec.reshape(n, tw, out_c).transpose(1, 0, 2)


# speedup vs baseline: 1.0349x; 1.0348x over previous
import jax
import jax.numpy as jnp
from jax import lax
from jax.experimental import pallas as pl
from jax.experimental.pallas import tpu as pltpu
from jax.experimental.pallas import tpu_sc as plsc

_EPS = 1e-5
_CHUNK = 128
_NW = 32


def _relu(x):
    return jnp.maximum(x, 0.0)


def _dot(a, b):
    return jnp.dot(a, b, preferred_element_type=jnp.float32)


def _ln(h, g, b):
    mu = jnp.mean(h, axis=-1, keepdims=True)
    var = jnp.mean((h - mu) ** 2, axis=-1, keepdims=True)
    return (h - mu) * lax.rsqrt(var + _EPS) * g + b


def _enc_body(x, W1, b1, W2, b2, g, be, o):
    h = _relu(_dot(x[...], W1[...]) + b1[...])
    h = _relu(_dot(h, W2[...]) + b2[...])
    o[...] = _ln(h, g[...], be[...])


def _pre_body(x, Ws, Wr, b1, a, b):
    xv = x[...]
    a[...] = _dot(xv, Ws[...]) + b1[...]
    b[...] = _dot(xv, Wr[...])


def _edge_body(c, e, We, W2, b2, g, be, eo):
    ev = e[...]
    h = _relu(c[...] + _dot(ev, We[...]))
    h = _relu(_dot(h, W2[...]) + b2[...])
    eo[...] = ev + _ln(h, g[...], be[...])


def _node_body(x, q0, q1, q2, q3, p0, p1, p2, p3,
               Wn, Wa, b1, W2, b2, g, be, o):
    xv = x[...]
    aggr = ((q0[...] + q1[...]) + (q2[...] + q3[...])
            - (p0[...] + p1[...]) - (p2[...] + p3[...]))
    h = _relu(_dot(xv, Wn[...]) + _dot(aggr, Wa[...]) + b1[...])
    h = _relu(_dot(h, W2[...]) + b2[...])
    o[...] = xv + _ln(h, g[...], be[...])


def _dec_body(x, W1, b1, W2, b2, dt, o):
    h = _dot(x[...], W1[...]) + b1[...]
    h = h * jax.nn.sigmoid(h)
    o[...] = (_dot(h, W2[...]) + b2[...]) * dt[...]


def _pick_tr(rows):
    for t in (2000, 1000, 500, 200, 100, 8):
        if rows % t == 0:
            return t
    return rows


def _rowwise(body, tiled_in, const_in, out_widths, out_dtype=jnp.float32):
    rows = tiled_in[0].shape[0]
    tr = _pick_tr(rows)
    grid = rows // tr
    in_specs = (
        [pl.BlockSpec((tr, a.shape[1]), lambda i: (i, 0)) for a in tiled_in]
        + [pl.BlockSpec(a.shape, lambda i, _nd=a.ndim: (0,) * _nd) for a in const_in]
    )
    out_shape = [jax.ShapeDtypeStruct((rows, w), out_dtype) for w in out_widths]
    out_specs = [pl.BlockSpec((tr, w), lambda i: (i, 0)) for w in out_widths]
    outs = pl.pallas_call(
        body,
        grid=(grid,),
        in_specs=in_specs,
        out_specs=out_specs,
        out_shape=out_shape,
    )(*tiled_in, *const_in)
    return outs


_SS = 256
_NBUF = 3


def _make_gather(n, e, l):
    nch = e // _SS
    groups = (((nch + _NW - 1) // _NW) + _NBUF - 1) // _NBUF
    mesh = plsc.VectorSubcoreMesh(core_axis_name="c", subcore_axis_name="s", num_cores=2, num_subcores=16)

    def body(a_hbm, b_hbm, snd_hbm, rcv_hbm, out_hbm,
             idx_s, idx_r, rows, sem_idx, sem_a, sem_b, sem_out):
        cid = lax.axis_index("c")
        sid = lax.axis_index("s")
        wid = sid * 2 + cid

        def issue_idx(k, b):
            c = wid + _NW * k

            @pl.when(c < nch)
            def _():
                base = c * _SS
                pltpu.async_copy(snd_hbm.at[pl.ds(base, _SS)], idx_s[b], sem_idx[b])
                pltpu.async_copy(rcv_hbm.at[pl.ds(base, _SS)], idx_r[b], sem_idx[b])

        for b in range(_NBUF):
            issue_idx(b, b)

        def issue_a(k, b):
            c = wid + _NW * k

            @pl.when(c < nch)
            def _():
                pltpu.make_async_copy(snd_hbm.at[pl.ds(c * _SS, _SS)], idx_s[b], sem_idx[b]).wait()
                for half in range(2):
                    isl = pl.ds(half * _CHUNK, _CHUNK)
                    pltpu.async_copy(a_hbm.at[idx_s[b].at[isl]],
                                     rows[b].at[isl, :], sem_a[b])

        def step(k, b):
            c = wid + _NW * k

            @pl.when(c < nch)
            def _():
                base = c * _SS
                pltpu.make_async_copy(rcv_hbm.at[pl.ds(base, _SS)], idx_r[b], sem_idx[b]).wait()
                for half in range(2):
                    isl = pl.ds(half * _CHUNK, _CHUNK)
                    pltpu.make_async_copy(a_hbm.at[idx_s[b].at[isl]],
                                          rows[b].at[isl, :], sem_a[b]).wait()
                ds = []
                for half in range(2):
                    isl = pl.ds(half * _CHUNK, _CHUNK)
                    ds.append(pltpu.async_copy(b_hbm.at[idx_r[b].at[isl]],
                                               rows[b].at[isl, :], sem_b[b], add=True))

                b1 = (b + 1) % _NBUF

                @pl.when((k + 1 >= _NBUF) & (wid + _NW * (k + 1) < nch))
                def _():
                    pltpu.make_async_copy(rows[b1], out_hbm.at[pl.ds(0, _SS), :], sem_out[b1]).wait()

                issue_a(k + 1, b1)

                for d in ds:
                    d.wait()
                issue_idx(k + _NBUF, b)
                pltpu.async_copy(rows[b], out_hbm.at[pl.ds(base, _SS), :], sem_out[b])

        def group(g, carry):
            for b in range(_NBUF):
                step(g * _NBUF + b, b)
            return carry

        issue_a(0, 0)
        lax.fori_loop(0, groups, group, 0)
        for b in range(_NBUF):
            pltpu.make_async_copy(rows[b], out_hbm.at[pl.ds(0, _SS), :], sem_out[b]).wait()

    return pl.kernel(
        body,
        out_type=jax.ShapeDtypeStruct((e, l), jnp.float32),
        mesh=mesh,
        scratch_types=[
            [pltpu.VMEM((_SS,), jnp.int32) for _ in range(_NBUF)],
            [pltpu.VMEM((_SS,), jnp.int32) for _ in range(_NBUF)],
            [pltpu.VMEM((_SS, l), jnp.float32) for _ in range(_NBUF)],
            [pltpu.SemaphoreType.DMA for _ in range(_NBUF)],
            [pltpu.SemaphoreType.DMA for _ in range(_NBUF)],
            [pltpu.SemaphoreType.DMA for _ in range(_NBUF)],
            [pltpu.SemaphoreType.DMA for _ in range(_NBUF)],
        ],
    )


def _make_scatter(n, e, l):
    rpt = ((n // 16) // 8) * 8
    last = n - 15 * rpt
    mesh = plsc.VectorSubcoreMesh(core_axis_name="c", subcore_axis_name="s", num_cores=2, num_subcores=16)

    def _striped_copy(sid, src, dst):
        @pl.when(sid < 15)
        def _():
            pltpu.sync_copy(src.at[pl.ds(sid * rpt, rpt), :],
                            dst.at[pl.ds(sid * rpt, rpt), :])

        @pl.when(sid == 15)
        def _():
            pltpu.sync_copy(src.at[pl.ds(15 * rpt, last), :],
                            dst.at[pl.ds(15 * rpt, last), :])

    nch = e // _CHUNK
    nbuf = 3
    groups = (((nch + _NW - 1) // _NW) + nbuf - 1) // nbuf

    def body(ne_hbm, rcv_hbm, zero_hbm, out_hbm, idx_r, rows, sem_in, sem_sc, acc):
        cid = lax.axis_index("c")
        sid = lax.axis_index("s")
        wid = sid * 2 + cid

        def issue_in(k, b):
            c = wid + _NW * k

            @pl.when(c < nch)
            def _():
                base = c * _CHUNK
                pltpu.async_copy(rcv_hbm.at[pl.ds(base, _CHUNK)], idx_r[b], sem_in[b])
                pltpu.async_copy(ne_hbm.at[pl.ds(base, _CHUNK), :], rows[b], sem_in[b])

        for b in range(2):
            issue_in(b, b)

        _striped_copy(sid, zero_hbm, acc)
        plsc.subcore_barrier()

        def step(k, b):
            c = wid + _NW * k

            @pl.when(c < nch)
            def _():
                base = c * _CHUNK
                pltpu.make_async_copy(rcv_hbm.at[pl.ds(base, _CHUNK)], idx_r[b], sem_in[b]).wait()
                pltpu.make_async_copy(ne_hbm.at[pl.ds(base, _CHUNK), :], rows[b], sem_in[b]).wait()
                pltpu.async_copy(rows[b], acc.at[idx_r[b]], sem_sc[b], add=True)
                bp = (b + 2) % nbuf

                @pl.when(k >= 1)
                def _():
                    pltpu.make_async_copy(rows[bp], acc.at[idx_r[bp]], sem_sc[bp]).wait()

                issue_in(k + 2, bp)

        def group(g, carry):
            for b in range(nbuf):
                step(g * nbuf + b, b)
            return carry

        lax.fori_loop(0, groups, group, 0)
        m = (nch - wid + _NW - 1) // _NW
        for b in range(nbuf):
            @pl.when((m - 1) % nbuf == b)
            def _(b=b):
                pltpu.make_async_copy(rows[b], acc.at[idx_r[b]], sem_sc[b]).wait()

        plsc.subcore_barrier()
        _striped_copy(sid, acc, out_hbm.at[cid])

    return pl.kernel(
        body,
        out_type=jax.ShapeDtypeStruct((2, n, l), jnp.float32),
        mesh=mesh,
        scratch_types=[
            [pltpu.VMEM((_CHUNK,), jnp.int32) for _ in range(3)],
            [pltpu.VMEM((_CHUNK, l), jnp.float32) for _ in range(3)],
            [pltpu.SemaphoreType.DMA for _ in range(3)],
            [pltpu.SemaphoreType.DMA for _ in range(3)],
            pltpu.VMEM_SHARED((n, l), jnp.float32),
        ],
    )


def kernel(node_features, mesh_edge_features, senders, receivers, params):
    p = params
    n, l = node_features.shape
    e = senders.shape[0]
    s_blocks = p["blk_e_W1"].shape[0]

    def r2(v):
        return v.reshape(1, -1)

    node_lat, = _rowwise(
        _enc_body, [node_features],
        [p["enc_n_W1"], r2(p["enc_n_b1"]), p["enc_n_W2"], r2(p["enc_n_b2"]),
         r2(p["enc_n_g"]), r2(p["enc_n_be"])], [l])

    eh = e // 2
    snd_h = (senders[:eh], senders[eh:])
    rcv_h = (receivers[:eh], receivers[eh:])
    edge_lat_h = []
    for h in range(2):
        el, = _rowwise(
            _enc_body, [mesh_edge_features[h * eh:(h + 1) * eh]],
            [p["enc_e_W1"], r2(p["enc_e_b1"]), p["enc_e_W2"], r2(p["enc_e_b2"]),
             r2(p["enc_e_g"]), r2(p["enc_e_be"])], [l])
        edge_lat_h.append(el)

    gather = _make_gather(n, eh, l)
    scatter = _make_scatter(n, eh, l)
    zeros_nl = jnp.zeros((n, l), jnp.float32)

    prev_h = [scatter(edge_lat_h[h], rcv_h[h], zeros_nl) for h in range(2)]
    for s in range(s_blocks):
        w1 = p["blk_e_W1"][s]
        a_tab, b_tab = _rowwise(
            _pre_body, [node_lat],
            [w1[:l], w1[l:2 * l], r2(p["blk_e_b1"][s])], [l, l])
        cur_h = []
        for h in range(2):
            c = gather(a_tab, b_tab, snd_h[h], rcv_h[h])
            edge_lat_h[h], = _rowwise(
                _edge_body, [c, edge_lat_h[h]],
                [w1[2 * l:], p["blk_e_W2"][s], r2(p["blk_e_b2"][s]),
                 r2(p["blk_e_g"][s]), r2(p["blk_e_be"][s])], [l])
            cur_h.append(scatter(edge_lat_h[h], rcv_h[h], zeros_nl))
        wn1 = p["blk_n_W1"][s]
        node_lat, = _rowwise(
            _node_body, [node_lat, cur_h[0][0], cur_h[0][1],
                         cur_h[1][0], cur_h[1][1],
                         prev_h[0][0], prev_h[0][1],
                         prev_h[1][0], prev_h[1][1]],
            [wn1[:l], wn1[l:], r2(p["blk_n_b1"][s]), p["blk_n_W2"][s],
             r2(p["blk_n_b2"][s]), r2(p["blk_n_g"][s]), r2(p["blk_n_be"][s])],
            [l])
        prev_h = cur_h

    tw_out = p["dec_b2"].shape[0]
    tw = 5
    out_c = tw_out // tw
    dt = jnp.repeat(jnp.arange(1, tw + 1), out_c).astype(jnp.float32)
    dec, = _rowwise(
        _dec_body, [node_lat],
        [p["dec_W1"], r2(p["dec_b1"]), p["dec_W2"], r2(p["dec_b2"]), r2(dt)],
        [tw_out])
    return dec.reshape(n, tw, out_c).transpose(1, 0, 2)
